# baseline (speedup 1.0000x reference)
import functools

import jax
import jax.numpy as jnp
from jax.experimental import pallas as pl
from jax.experimental.pallas import tpu as pltpu


def _row_chunk(ho, wo, k):
    budget = 2 * 1024 * 1024
    tr = max(1, budget // (wo * k * 4))
    tr = min(tr, ho)
    while ho % tr:
        tr -= 1
    return tr


def _wmat3(w):
    return w.transpose(2, 3, 1, 0).reshape(9 * w.shape[1], w.shape[0])


def _zero_border_nhwc(ref, s, c):
    ref[0:1] = jnp.zeros((1, s + 2, c), jnp.float32)
    ref[s + 1:s + 2] = jnp.zeros((1, s + 2, c), jnp.float32)
    ref[:, 0:1] = jnp.zeros((s + 2, 1, c), jnp.float32)
    ref[:, s + 1:s + 2] = jnp.zeros((s + 2, 1, c), jnp.float32)


def _conv_from(src_ref, wm_ref, b_ref, s, cin, cout):
    k = 9 * cin
    tr = _row_chunk(s, s, k)
    for r in range(0, s, tr):
        parts = [src_ref[r + kh:r + kh + tr, kw:kw + s, :]
                 for kh in range(3) for kw in range(3)]
        xs = jnp.concatenate(parts, axis=-1).reshape(tr * s, k)
        y = jnp.dot(xs, wm_ref[:], preferred_element_type=jnp.float32)
        y = (y + b_ref[0]).reshape(tr, s, cout)
        yield r, tr, y


def _chain_body(x_ref, wf_ref, bf_ref, w1_ref, b1_ref, w2_ref, b2_ref,
                w3_ref, b3_ref, w4_ref, b4_ref, o_ref, e_ref, a_ref, h_ref,
                *, s, cin):
    _zero_border_nhwc(a_ref, s, 128)
    _zero_border_nhwc(h_ref, s, 64)
    _zero_border_nhwc(o_ref.at[0], s, 128)
    xp = x_ref.at[0]
    for r, tr, y in _conv_from(xp, wf_ref, bf_ref, s, cin, 128):
        a_ref[1 + r:1 + r + tr, 1:1 + s] = y
    for r, tr, y in _conv_from(a_ref, w1_ref, b1_ref, s, 128, 64):
        h_ref[1 + r:1 + r + tr, 1:1 + s] = jnp.maximum(y, 0.0)
    for r, tr, y in _conv_from(h_ref, w2_ref, b2_ref, s, 64, 128):
        a_ref[1 + r:1 + r + tr, 1:1 + s] = a_ref[1 + r:1 + r + tr, 1:1 + s] + y
    for r, tr, y in _conv_from(a_ref, w3_ref, b3_ref, s, 128, 64):
        h_ref[1 + r:1 + r + tr, 1:1 + s] = jnp.maximum(y, 0.0)
    for r, tr, y in _conv_from(h_ref, w4_ref, b4_ref, s, 64, 128):
        t = jnp.maximum(a_ref[1 + r:1 + r + tr, 1:1 + s] + y, 0.0)
        o_ref[0, 1 + r:1 + r + tr, 1:1 + s] = t
        e_ref[0, :, r:r + tr] = (jnp.transpose(t.reshape(tr * s, 128))
                                 .reshape(128, tr, s))


def _res_chain(hpad, wf, bf, res_params):
    n, sp, _, cin = hpad.shape
    s = sp - 2
    (w1, b1, w2, b2), (w3, b3, w4, b4) = res_params
    mats = [_wmat3(wf), bf.reshape(1, 128), _wmat3(w1), b1.reshape(1, 64),
            _wmat3(w2), b2.reshape(1, 128), _wmat3(w3), b3.reshape(1, 64),
            _wmat3(w4), b4.reshape(1, 128)]
    in_specs = [pl.BlockSpec((1, sp, sp, cin), lambda i: (i, 0, 0, 0))]
    for m in mats:
        in_specs.append(pl.BlockSpec(m.shape, lambda i: (0, 0)))
    body = functools.partial(_chain_body, s=s, cin=cin)
    return pl.pallas_call(
        body,
        grid=(n,),
        in_specs=in_specs,
        out_specs=[pl.BlockSpec((1, sp, sp, 128), lambda i: (i, 0, 0, 0)),
                   pl.BlockSpec((1, 128, s, s), lambda i: (i, 0, 0, 0))],
        out_shape=[jax.ShapeDtypeStruct((n, sp, sp, 128), jnp.float32),
                   jax.ShapeDtypeStruct((n, 128, s, s), jnp.float32)],
        scratch_shapes=[pltpu.VMEM((sp, sp, 128), jnp.float32),
                        pltpu.VMEM((sp, sp, 64), jnp.float32)],
    )(hpad, *mats)


def _down_body(x_ref, w_ref, b_ref, o_ref, *, ho, wo, cin, cout):
    _zero_border_nhwc(o_ref.at[0], ho, cout)
    k = 16 * cin
    tr = _row_chunk(ho, wo, k)
    for r in range(0, ho, tr):
        parts = [x_ref[0, r + kh // 2:r + kh // 2 + tr, kh % 2,
                       kw // 2:kw // 2 + wo,
                       (kw % 2) * cin:(kw % 2) * cin + cin]
                 for kh in range(4) for kw in range(4)]
        xs = jnp.concatenate(parts, axis=-1).reshape(tr * wo, k)
        y = jnp.dot(xs, w_ref[:], preferred_element_type=jnp.float32)
        y = (y + b_ref[0]).reshape(tr, wo, cout)
        o_ref[0, 1 + r:1 + r + tr, 1:1 + wo] = jnp.maximum(y, 0.0)


def _down_conv(hpad, w, b):
    n, hp2, _, c = hpad.shape
    hh = hp2 - 2
    ho = hh // 2
    o = w.shape[0]
    z = hpad.reshape(n, ho + 1, 2, ho + 1, 2 * c)
    wmat = w.transpose(2, 3, 1, 0).reshape(16 * c, o)
    body = functools.partial(_down_body, ho=ho, wo=ho, cin=c, cout=o)
    return pl.pallas_call(
        body,
        grid=(n,),
        in_specs=[
            pl.BlockSpec((1, ho + 1, 2, ho + 1, 2 * c),
                         lambda i: (i, 0, 0, 0, 0)),
            pl.BlockSpec(wmat.shape, lambda i: (0, 0)),
            pl.BlockSpec((1, o), lambda i: (0, 0)),
        ],
        out_specs=pl.BlockSpec((1, ho + 2, ho + 2, o), lambda i: (i, 0, 0, 0)),
        out_shape=jax.ShapeDtypeStruct((n, ho + 2, ho + 2, o), jnp.float32),
    )(z, wmat, b.reshape(1, o))


def _first_body(x_ref, p_ref, w_ref, b_ref, o_ref, *, ho, wo, cin):
    wq = wo + 1
    k = 16 * cin
    g = 4 * cin
    _zero_border_nhwc(o_ref.at[0], ho, 64)
    tr = 16
    for r in range(0, ho, tr):
        slabs = [x_ref[0, c, r:r + tr + 1, sh * 2 * wq:(sh + 1) * 2 * wq]
                 for c in range(cin) for sh in range(2)]
        cat = jnp.concatenate(slabs, axis=-1)
        c6 = cat.reshape(tr + 1, 2 * cin, 2 * wq)
        zc = jnp.transpose(c6, (0, 2, 1)).reshape(tr + 1, 2 * wq * 2 * cin)
        parts = [zc[ph:ph + tr, pw * g:pw * g + wo * g].reshape(tr, wo, g)
                 for ph in range(2) for pw in range(2)]
        xs = jnp.concatenate(parts, axis=-1).reshape(tr * wo, k)
        xs = jnp.dot(xs, p_ref[:], preferred_element_type=jnp.float32)
        y = jnp.dot(xs, w_ref[:], preferred_element_type=jnp.float32)
        y = (y + b_ref[0]).reshape(tr, wo, 64)
        o_ref[0, 1 + r:1 + r + tr, 1:1 + wo] = jnp.maximum(y, 0.0)


def _first_down_conv(x, w, b):
    n, c, hh, _ = x.shape
    ho = hh // 2
    k = 16 * c
    xp = jnp.pad(x, ((0, 0), (0, 0), (1, 1), (1, 1)))
    z = xp.reshape(n, c, ho + 1, 4 * (ho + 1))
    pm = [[0.0] * k for _ in range(k)]
    for ph in range(2):
        for pw in range(2):
            for ci in range(c):
                for sh in range(2):
                    for sw in range(2):
                        src = (ph * 2 + pw) * 4 * c + (sw * c + ci) * 2 + sh
                        dst = ((2 * ph + sh) * 4 + 2 * pw + sw) * c + ci
                        pm[src][dst] = 1.0
    pmat = jnp.asarray(pm, dtype=jnp.float32)
    w2 = w.transpose(2, 3, 1, 0).reshape(k, 64)
    body = functools.partial(_first_body, ho=ho, wo=ho, cin=c)
    return pl.pallas_call(
        body,
        grid=(n,),
        in_specs=[
            pl.BlockSpec((1, c, ho + 1, 4 * (ho + 1)), lambda i: (i, 0, 0, 0)),
            pl.BlockSpec((k, k), lambda i: (0, 0)),
            pl.BlockSpec((k, 64), lambda i: (0, 0)),
            pl.BlockSpec((1, 64), lambda i: (0, 0)),
        ],
        out_specs=pl.BlockSpec((1, ho + 2, ho + 2, 64), lambda i: (i, 0, 0, 0)),
        out_shape=jax.ShapeDtypeStruct((n, ho + 2, ho + 2, 64), jnp.float32),
    )(z, pmat, w2, b.reshape(1, 64))


def kernel(x, params):
    e = []
    h = None
    for pi, p in enumerate(params):
        for (w, b) in p['down']:
            if h is None:
                h = _first_down_conv(x, w, b)
            else:
                h = _down_conv(h, w, b)
        wf, bf = p['final']
        h, e_nchw = _res_chain(h, wf, bf, p['res'])
        e.append(e_nchw)
    return tuple(e)


# speedup vs baseline: 1.0883x; 1.0883x over previous
import functools

import jax
import jax.numpy as jnp
from jax.experimental import pallas as pl
from jax.experimental.pallas import tpu as pltpu


def _row_chunk(ho, wo, k):
    budget = 2 * 1024 * 1024
    tr = max(1, budget // (wo * k * 4))
    tr = min(tr, ho)
    while ho % tr:
        tr -= 1
    return tr


def _wmat3(w):
    return w.transpose(2, 3, 1, 0).reshape(9 * w.shape[1], w.shape[0])


def _zero_border_nhwc(ref, s, c):
    ref[0:1] = jnp.zeros((1, s + 2, c), jnp.float32)
    ref[s + 1:s + 2] = jnp.zeros((1, s + 2, c), jnp.float32)
    ref[:, 0:1] = jnp.zeros((s + 2, 1, c), jnp.float32)
    ref[:, s + 1:s + 2] = jnp.zeros((s + 2, 1, c), jnp.float32)


def _conv_from(src_ref, wm_ref, b_ref, s, cin, cout):
    k = 9 * cin
    tr = _row_chunk(s, s, k)
    for r in range(0, s, tr):
        parts = [src_ref[r + kh:r + kh + tr, kw:kw + s, :]
                 for kh in range(3) for kw in range(3)]
        xs = jnp.concatenate(parts, axis=-1).reshape(tr * s, k)
        y = jnp.dot(xs, wm_ref[:], preferred_element_type=jnp.float32)
        y = (y + b_ref[0]).reshape(tr, s, cout)
        yield r, tr, y


def _chain_body(x_ref, wf_ref, bf_ref, w1_ref, b1_ref, w2_ref, b2_ref,
                w3_ref, b3_ref, w4_ref, b4_ref, o_ref, a_ref, h_ref,
                *, s, cin):
    _zero_border_nhwc(a_ref, s, 128)
    _zero_border_nhwc(h_ref, s, 64)
    _zero_border_nhwc(o_ref.at[0], s, 128)
    xp = x_ref.at[0]
    for r, tr, y in _conv_from(xp, wf_ref, bf_ref, s, cin, 128):
        a_ref[1 + r:1 + r + tr, 1:1 + s] = y
    for r, tr, y in _conv_from(a_ref, w1_ref, b1_ref, s, 128, 64):
        h_ref[1 + r:1 + r + tr, 1:1 + s] = jnp.maximum(y, 0.0)
    for r, tr, y in _conv_from(h_ref, w2_ref, b2_ref, s, 64, 128):
        a_ref[1 + r:1 + r + tr, 1:1 + s] = a_ref[1 + r:1 + r + tr, 1:1 + s] + y
    for r, tr, y in _conv_from(a_ref, w3_ref, b3_ref, s, 128, 64):
        h_ref[1 + r:1 + r + tr, 1:1 + s] = jnp.maximum(y, 0.0)
    for r, tr, y in _conv_from(h_ref, w4_ref, b4_ref, s, 64, 128):
        o_ref[0, 1 + r:1 + r + tr, 1:1 + s] = jnp.maximum(
            a_ref[1 + r:1 + r + tr, 1:1 + s] + y, 0.0)


def _res_chain(hpad, wf, bf, res_params):
    n, sp, _, cin = hpad.shape
    s = sp - 2
    (w1, b1, w2, b2), (w3, b3, w4, b4) = res_params
    mats = [_wmat3(wf), bf.reshape(1, 128), _wmat3(w1), b1.reshape(1, 64),
            _wmat3(w2), b2.reshape(1, 128), _wmat3(w3), b3.reshape(1, 64),
            _wmat3(w4), b4.reshape(1, 128)]
    in_specs = [pl.BlockSpec((1, sp, sp, cin), lambda i: (i, 0, 0, 0))]
    for m in mats:
        in_specs.append(pl.BlockSpec(m.shape, lambda i: (0, 0)))
    body = functools.partial(_chain_body, s=s, cin=cin)
    return pl.pallas_call(
        body,
        grid=(n,),
        compiler_params=pltpu.CompilerParams(
            dimension_semantics=("parallel",)),
        in_specs=in_specs,
        out_specs=pl.BlockSpec((1, sp, sp, 128), lambda i: (i, 0, 0, 0)),
        out_shape=jax.ShapeDtypeStruct((n, sp, sp, 128), jnp.float32),
        scratch_shapes=[pltpu.VMEM((sp, sp, 128), jnp.float32),
                        pltpu.VMEM((sp, sp, 64), jnp.float32)],
    )(hpad, *mats)


def _down_body(x_ref, w_ref, b_ref, o_ref, *, ho, wo, cin, cout):
    _zero_border_nhwc(o_ref.at[0], ho, cout)
    k = 16 * cin
    tr = _row_chunk(ho, wo, k)
    for r in range(0, ho, tr):
        parts = [x_ref[0, r + kh // 2:r + kh // 2 + tr, kh % 2,
                       kw // 2:kw // 2 + wo,
                       (kw % 2) * cin:(kw % 2) * cin + cin]
                 for kh in range(4) for kw in range(4)]
        xs = jnp.concatenate(parts, axis=-1).reshape(tr * wo, k)
        y = jnp.dot(xs, w_ref[:], preferred_element_type=jnp.float32)
        y = (y + b_ref[0]).reshape(tr, wo, cout)
        o_ref[0, 1 + r:1 + r + tr, 1:1 + wo] = jnp.maximum(y, 0.0)


def _down_conv(hpad, w, b):
    n, hp2, _, c = hpad.shape
    hh = hp2 - 2
    ho = hh // 2
    o = w.shape[0]
    z = hpad.reshape(n, ho + 1, 2, ho + 1, 2 * c)
    wmat = w.transpose(2, 3, 1, 0).reshape(16 * c, o)
    body = functools.partial(_down_body, ho=ho, wo=ho, cin=c, cout=o)
    return pl.pallas_call(
        body,
        grid=(n,),
        compiler_params=pltpu.CompilerParams(
            dimension_semantics=("parallel",)),
        in_specs=[
            pl.BlockSpec((1, ho + 1, 2, ho + 1, 2 * c),
                         lambda i: (i, 0, 0, 0, 0)),
            pl.BlockSpec(wmat.shape, lambda i: (0, 0)),
            pl.BlockSpec((1, o), lambda i: (0, 0)),
        ],
        out_specs=pl.BlockSpec((1, ho + 2, ho + 2, o), lambda i: (i, 0, 0, 0)),
        out_shape=jax.ShapeDtypeStruct((n, ho + 2, ho + 2, o), jnp.float32),
    )(z, wmat, b.reshape(1, o))


def _first_body(x_ref, p_ref, w_ref, b_ref, o_ref, *, ho, wo, cin):
    wq = wo + 1
    k = 16 * cin
    g = 4 * cin
    _zero_border_nhwc(o_ref.at[0], ho, 64)
    tr = 16
    for r in range(0, ho, tr):
        slabs = [x_ref[0, c, r:r + tr + 1, sh * 2 * wq:(sh + 1) * 2 * wq]
                 for c in range(cin) for sh in range(2)]
        cat = jnp.concatenate(slabs, axis=-1)
        c6 = cat.reshape(tr + 1, 2 * cin, 2 * wq)
        zc = jnp.transpose(c6, (0, 2, 1)).reshape(tr + 1, 2 * wq * 2 * cin)
        parts = [zc[ph:ph + tr, pw * g:pw * g + wo * g].reshape(tr, wo, g)
                 for ph in range(2) for pw in range(2)]
        xs = jnp.concatenate(parts, axis=-1).reshape(tr * wo, k)
        xs = jnp.dot(xs, p_ref[:], preferred_element_type=jnp.float32)
        y = jnp.dot(xs, w_ref[:], preferred_element_type=jnp.float32)
        y = (y + b_ref[0]).reshape(tr, wo, 64)
        o_ref[0, 1 + r:1 + r + tr, 1:1 + wo] = jnp.maximum(y, 0.0)


def _first_down_conv(x, w, b):
    n, c, hh, _ = x.shape
    ho = hh // 2
    k = 16 * c
    xp = jnp.pad(x, ((0, 0), (0, 0), (1, 1), (1, 1)))
    z = xp.reshape(n, c, ho + 1, 4 * (ho + 1))
    pm = [[0.0] * k for _ in range(k)]
    for ph in range(2):
        for pw in range(2):
            for ci in range(c):
                for sh in range(2):
                    for sw in range(2):
                        src = (ph * 2 + pw) * 4 * c + (sw * c + ci) * 2 + sh
                        dst = ((2 * ph + sh) * 4 + 2 * pw + sw) * c + ci
                        pm[src][dst] = 1.0
    pmat = jnp.asarray(pm, dtype=jnp.float32)
    w2 = w.transpose(2, 3, 1, 0).reshape(k, 64)
    body = functools.partial(_first_body, ho=ho, wo=ho, cin=c)
    return pl.pallas_call(
        body,
        grid=(n,),
        compiler_params=pltpu.CompilerParams(
            dimension_semantics=("parallel",)),
        in_specs=[
            pl.BlockSpec((1, c, ho + 1, 4 * (ho + 1)), lambda i: (i, 0, 0, 0)),
            pl.BlockSpec((k, k), lambda i: (0, 0)),
            pl.BlockSpec((k, 64), lambda i: (0, 0)),
            pl.BlockSpec((1, 64), lambda i: (0, 0)),
        ],
        out_specs=pl.BlockSpec((1, ho + 2, ho + 2, 64), lambda i: (i, 0, 0, 0)),
        out_shape=jax.ShapeDtypeStruct((n, ho + 2, ho + 2, 64), jnp.float32),
    )(z, pmat, w2, b.reshape(1, 64))


def kernel(x, params):
    e = []
    h = None
    for pi, p in enumerate(params):
        for (w, b) in p['down']:
            if h is None:
                h = _first_down_conv(x, w, b)
            else:
                h = _down_conv(h, w, b)
        wf, bf = p['final']
        h = _res_chain(h, wf, bf, p['res'])
        e.append(jnp.transpose(h[:, 1:-1, 1:-1, :], (0, 3, 1, 2)))
    return tuple(e)


# speedup vs baseline: 1.1139x; 1.0235x over previous
import functools

import jax
import jax.numpy as jnp
from jax.experimental import pallas as pl
from jax.experimental.pallas import tpu as pltpu


def _row_chunk(ho, wo, k):
    budget = 2 * 1024 * 1024
    tr = max(1, budget // (wo * k * 4))
    tr = min(tr, ho)
    while ho % tr:
        tr -= 1
    return tr


def _wmat3(w):
    return w.transpose(2, 3, 1, 0).reshape(9 * w.shape[1], w.shape[0])


def _zero_border_nhwc(ref, s, c):
    ref[0:1] = jnp.zeros((1, s + 2, c), jnp.float32)
    ref[s + 1:s + 2] = jnp.zeros((1, s + 2, c), jnp.float32)
    ref[:, 0:1] = jnp.zeros((s + 2, 1, c), jnp.float32)
    ref[:, s + 1:s + 2] = jnp.zeros((s + 2, 1, c), jnp.float32)


def _conv_from(src_ref, wm_ref, b_ref, s, cin, cout):
    k = 9 * cin
    tr = _row_chunk(s, s, k)
    for r in range(0, s, tr):
        parts = [src_ref[r + kh:r + kh + tr, kw:kw + s, :]
                 for kh in range(3) for kw in range(3)]
        xs = jnp.concatenate(parts, axis=-1).reshape(tr * s, k)
        y = jnp.dot(xs, wm_ref[:], preferred_element_type=jnp.float32)
        y = (y + b_ref[0]).reshape(tr, s, cout)
        yield r, tr, y


def _chain_body(x_ref, wf_ref, bf_ref, w1_ref, b1_ref, w2_ref, b2_ref,
                w3_ref, b3_ref, w4_ref, b4_ref, o_ref, a_ref, h_ref,
                *, s, cin):
    _zero_border_nhwc(a_ref, s, 128)
    _zero_border_nhwc(h_ref, s, 64)
    _zero_border_nhwc(o_ref.at[0], s, 128)
    xp = x_ref.at[0]
    for r, tr, y in _conv_from(xp, wf_ref, bf_ref, s, cin, 128):
        a_ref[1 + r:1 + r + tr, 1:1 + s] = y
    for r, tr, y in _conv_from(a_ref, w1_ref, b1_ref, s, 128, 64):
        h_ref[1 + r:1 + r + tr, 1:1 + s] = jnp.maximum(y, 0.0)
    for r, tr, y in _conv_from(h_ref, w2_ref, b2_ref, s, 64, 128):
        a_ref[1 + r:1 + r + tr, 1:1 + s] = a_ref[1 + r:1 + r + tr, 1:1 + s] + y
    for r, tr, y in _conv_from(a_ref, w3_ref, b3_ref, s, 128, 64):
        h_ref[1 + r:1 + r + tr, 1:1 + s] = jnp.maximum(y, 0.0)
    for r, tr, y in _conv_from(h_ref, w4_ref, b4_ref, s, 64, 128):
        o_ref[0, 1 + r:1 + r + tr, 1:1 + s] = jnp.maximum(
            a_ref[1 + r:1 + r + tr, 1:1 + s] + y, 0.0)


def _res_chain(hpad, wf, bf, res_params):
    n, sp, _, cin = hpad.shape
    s = sp - 2
    (w1, b1, w2, b2), (w3, b3, w4, b4) = res_params
    mats = [_wmat3(wf), bf.reshape(1, 128), _wmat3(w1), b1.reshape(1, 64),
            _wmat3(w2), b2.reshape(1, 128), _wmat3(w3), b3.reshape(1, 64),
            _wmat3(w4), b4.reshape(1, 128)]
    in_specs = [pl.BlockSpec((1, sp, sp, cin), lambda i: (i, 0, 0, 0))]
    for m in mats:
        in_specs.append(pl.BlockSpec(m.shape, lambda i: (0, 0)))
    body = functools.partial(_chain_body, s=s, cin=cin)
    return pl.pallas_call(
        body,
        grid=(n,),
        compiler_params=pltpu.CompilerParams(
            dimension_semantics=("parallel",)),
        in_specs=in_specs,
        out_specs=pl.BlockSpec((1, sp, sp, 128), lambda i: (i, 0, 0, 0)),
        out_shape=jax.ShapeDtypeStruct((n, sp, sp, 128), jnp.float32),
        scratch_shapes=[pltpu.VMEM((sp, sp, 128), jnp.float32),
                        pltpu.VMEM((sp, sp, 64), jnp.float32)],
    )(hpad, *mats)


def _down_body(x_ref, w_ref, b_ref, o_ref, *, ho, wo, cin, cout):
    _zero_border_nhwc(o_ref.at[0], ho, cout)
    k = 16 * cin
    tr = _row_chunk(ho, wo, k)
    for r in range(0, ho, tr):
        parts = [x_ref[0, r + kh // 2:r + kh // 2 + tr, kh % 2,
                       kw // 2:kw // 2 + wo,
                       (kw % 2) * cin:(kw % 2) * cin + cin]
                 for kh in range(4) for kw in range(4)]
        xs = jnp.concatenate(parts, axis=-1).reshape(tr * wo, k)
        y = jnp.dot(xs, w_ref[:], preferred_element_type=jnp.float32)
        y = (y + b_ref[0]).reshape(tr, wo, cout)
        o_ref[0, 1 + r:1 + r + tr, 1:1 + wo] = jnp.maximum(y, 0.0)


def _down_conv(hpad, w, b):
    n, hp2, _, c = hpad.shape
    hh = hp2 - 2
    ho = hh // 2
    o = w.shape[0]
    z = hpad.reshape(n, ho + 1, 2, ho + 1, 2 * c)
    wmat = w.transpose(2, 3, 1, 0).reshape(16 * c, o)
    body = functools.partial(_down_body, ho=ho, wo=ho, cin=c, cout=o)
    return pl.pallas_call(
        body,
        grid=(n,),
        compiler_params=pltpu.CompilerParams(
            dimension_semantics=("parallel",)),
        in_specs=[
            pl.BlockSpec((1, ho + 1, 2, ho + 1, 2 * c),
                         lambda i: (i, 0, 0, 0, 0)),
            pl.BlockSpec(wmat.shape, lambda i: (0, 0)),
            pl.BlockSpec((1, o), lambda i: (0, 0)),
        ],
        out_specs=pl.BlockSpec((1, ho + 2, ho + 2, o), lambda i: (i, 0, 0, 0)),
        out_shape=jax.ShapeDtypeStruct((n, ho + 2, ho + 2, o), jnp.float32),
    )(z, wmat, b.reshape(1, o))


def _first_body(x_ref, p_ref, w_ref, b_ref, o_ref, *, ho, wo, cin):
    wq = wo + 1
    k = 16 * cin
    g = 4 * cin
    _zero_border_nhwc(o_ref.at[0], ho, 64)
    tr = 32
    for r in range(0, ho, tr):
        slabs = [x_ref[0, c, r:r + tr + 1, sh * 2 * wq:(sh + 1) * 2 * wq]
                 for c in range(cin) for sh in range(2)]
        cat = jnp.concatenate(slabs, axis=-1)
        c6 = cat.reshape(tr + 1, 2 * cin, 2 * wq)
        zc = jnp.transpose(c6, (0, 2, 1)).reshape(tr + 1, 2 * wq * 2 * cin)
        parts = [zc[ph:ph + tr, pw * g:pw * g + wo * g].reshape(tr, wo, g)
                 for ph in range(2) for pw in range(2)]
        xs = jnp.concatenate(parts, axis=-1).reshape(tr * wo, k)
        xs = jnp.dot(xs, p_ref[:], preferred_element_type=jnp.float32)
        y = jnp.dot(xs, w_ref[:], preferred_element_type=jnp.float32)
        y = (y + b_ref[0]).reshape(tr, wo, 64)
        o_ref[0, 1 + r:1 + r + tr, 1:1 + wo] = jnp.maximum(y, 0.0)


def _first_down_conv(x, w, b):
    n, c, hh, _ = x.shape
    ho = hh // 2
    k = 16 * c
    xp = jnp.pad(x, ((0, 0), (0, 0), (1, 1), (1, 1)))
    z = xp.reshape(n, c, ho + 1, 4 * (ho + 1))
    pm = [[0.0] * k for _ in range(k)]
    for ph in range(2):
        for pw in range(2):
            for ci in range(c):
                for sh in range(2):
                    for sw in range(2):
                        src = (ph * 2 + pw) * 4 * c + (sw * c + ci) * 2 + sh
                        dst = ((2 * ph + sh) * 4 + 2 * pw + sw) * c + ci
                        pm[src][dst] = 1.0
    pmat = jnp.asarray(pm, dtype=jnp.float32)
    w2 = w.transpose(2, 3, 1, 0).reshape(k, 64)
    body = functools.partial(_first_body, ho=ho, wo=ho, cin=c)
    return pl.pallas_call(
        body,
        grid=(n,),
        compiler_params=pltpu.CompilerParams(
            dimension_semantics=("parallel",)),
        in_specs=[
            pl.BlockSpec((1, c, ho + 1, 4 * (ho + 1)), lambda i: (i, 0, 0, 0)),
            pl.BlockSpec((k, k), lambda i: (0, 0)),
            pl.BlockSpec((k, 64), lambda i: (0, 0)),
            pl.BlockSpec((1, 64), lambda i: (0, 0)),
        ],
        out_specs=pl.BlockSpec((1, ho + 2, ho + 2, 64), lambda i: (i, 0, 0, 0)),
        out_shape=jax.ShapeDtypeStruct((n, ho + 2, ho + 2, 64), jnp.float32),
    )(z, pmat, w2, b.reshape(1, 64))


def kernel(x, params):
    e = []
    h = None
    for pi, p in enumerate(params):
        for (w, b) in p['down']:
            if h is None:
                h = _first_down_conv(x, w, b)
            else:
                h = _down_conv(h, w, b)
        wf, bf = p['final']
        h = _res_chain(h, wf, bf, p['res'])
        e.append(jnp.transpose(h[:, 1:-1, 1:-1, :], (0, 3, 1, 2)))
    return tuple(e)
